# baseline (device time: 38175 ns/iter reference)
import jax
import jax.numpy as jnp
from jax import lax
from jax.experimental import pallas as pl
from jax.experimental.pallas import tpu as pltpu

_sem_signal = getattr(pl, "semaphore_signal", None) or pltpu.semaphore_signal
_sem_wait = getattr(pl, "semaphore_wait", None) or pltpu.semaphore_wait
_DevIdType = getattr(pl, "DeviceIdType", None) or pltpu.DeviceIdType
_CompilerParams = getattr(pltpu, "CompilerParams", None) or pltpu.TPUCompilerParams


def kernel(x, W):
    t, d = x.shape
    _, v_loc = W.shape

    def body(x_ref, w_ref, out_ref, send_ref, recv_ref, send_sem, recv_sem):
        my_x = lax.axis_index("x")
        my_y = lax.axis_index("y")
        nbr = (my_x, 1 - my_y)

        barrier_sem = pltpu.get_barrier_semaphore()
        _sem_signal(
            barrier_sem, inc=1, device_id=nbr, device_id_type=_DevIdType.MESH
        )
        _sem_wait(barrier_sem, 1)

        logits = jnp.dot(
            x_ref[...].astype(jnp.bfloat16),
            w_ref[...].astype(jnp.bfloat16),
            preferred_element_type=jnp.float32,
        )
        send_ref[...] = logits.astype(jnp.bfloat16)

        rdma = pltpu.make_async_remote_copy(
            src_ref=send_ref,
            dst_ref=recv_ref,
            send_sem=send_sem,
            recv_sem=recv_sem,
            device_id=nbr,
            device_id_type=_DevIdType.MESH,
        )
        rdma.start()

        m_mine = jnp.max(logits, axis=-1, keepdims=True)
        e_mine = jnp.exp(logits - m_mine)
        s_mine = jnp.sum(e_mine, axis=-1, keepdims=True)

        rdma.wait()

        other = recv_ref[...].astype(jnp.float32)
        m_other = jnp.max(other, axis=-1, keepdims=True)
        e_other = jnp.exp(other - m_other)
        s_other = jnp.sum(e_other, axis=-1, keepdims=True)

        m = jnp.maximum(m_mine, m_other)
        c_mine = jnp.exp(m_mine - m)
        c_other = jnp.exp(m_other - m)
        s = s_mine * c_mine + s_other * c_other
        p_mine = e_mine * (c_mine / s)
        p_other = e_other * (c_other / s)

        @pl.when(my_y == 0)
        def _():
            out_ref[:, :v_loc] = p_mine
            out_ref[:, v_loc:] = p_other

        @pl.when(my_y == 1)
        def _():
            out_ref[:, :v_loc] = p_other
            out_ref[:, v_loc:] = p_mine

    return pl.pallas_call(
        body,
        out_shape=jax.ShapeDtypeStruct((t, 2 * v_loc), jnp.float32),
        in_specs=[
            pl.BlockSpec(memory_space=pltpu.VMEM),
            pl.BlockSpec(memory_space=pltpu.VMEM),
        ],
        out_specs=pl.BlockSpec(memory_space=pltpu.VMEM),
        scratch_shapes=[
            pltpu.VMEM((t, v_loc), jnp.bfloat16),
            pltpu.VMEM((t, v_loc), jnp.bfloat16),
            pltpu.SemaphoreType.DMA,
            pltpu.SemaphoreType.DMA,
        ],
        compiler_params=_CompilerParams(collective_id=0),
    )(x, W)
